# Initial kernel scaffold; baseline (speedup 1.0000x reference)
import jax
import jax.numpy as jnp
from jax.experimental import pallas as pl


def kernel(numeric, emb_idx, lengths, tables, gamma, beta):
    raise NotImplementedError("write your pallas kernel here")


# speedup vs baseline: 8.6012x; 8.6012x over previous
import functools

import jax
import jax.numpy as jnp
from jax import lax
from jax.experimental import pallas as pl
from jax.experimental.pallas import tpu as pltpu
from jax.experimental.pallas import tpu_sc as plsc

B, T = 1024, 50
N_NUM, N_EMB = 13, 26
EMB_DIM = 32
VOCAB = 100000
F = N_NUM + N_EMB * EMB_DIM
EPS = 1e-5

BT = B * T

NC, NS = 2, 16
NW = NC * NS
TOKW = BT // NW


N_IDX = BT * N_EMB
PER_W = N_IDX // NW
CHUNK = 1600
N_CH = PER_W // CHUNK


def _sc_gather(flat_tables, flat_idx):
    mesh = plsc.VectorSubcoreMesh(core_axis_name="c", subcore_axis_name="s")

    @functools.partial(
        pl.kernel,
        mesh=mesh,
        out_type=jax.ShapeDtypeStruct((N_IDX, EMB_DIM), jnp.float32),
        scratch_types=[
            pltpu.VMEM((CHUNK,), jnp.int32),
            pltpu.VMEM((CHUNK, EMB_DIM), jnp.float32),
            pltpu.SemaphoreType.DMA,
        ],
        compiler_params=pltpu.CompilerParams(use_tc_tiling_on_sc=False),
    )
    def gather_k(tbl_hbm, idx_hbm, out_hbm, idx_v, rows_v, sem):
        wid = lax.axis_index("s") * NC + lax.axis_index("c")
        base = wid * PER_W

        def cbody(c, _):
            off = base + c * CHUNK
            pltpu.sync_copy(idx_hbm.at[pl.ds(off, CHUNK)], idx_v)
            pltpu.async_copy(tbl_hbm.at[idx_v], rows_v, sem).wait()
            pltpu.sync_copy(rows_v, out_hbm.at[pl.ds(off, CHUNK)])
            return 0

        lax.fori_loop(0, N_CH, cbody, 0)

    return gather_k(flat_tables, flat_idx)


BG = 16
NBLK = B // BG


def _stats_body(len_ref, feats_ref, gamma_ref, beta_ref,
                scale_ref, bias_ref, acc_sum, acc_sq, acc_n):
    i = pl.program_id(0)

    @pl.when(i == 0)
    def _():
        acc_sum[...] = jnp.zeros_like(acc_sum)
        acc_sq[...] = jnp.zeros_like(acc_sq)
        acc_n[...] = jnp.zeros_like(acc_n)

    lens = len_ref[...]
    m3 = (lax.broadcasted_iota(jnp.int32, (BG, T, F), 1)
          < lens.reshape(BG, 1, 1)).astype(jnp.float32)
    feats = feats_ref[...]
    fm = feats * m3
    acc_sum[...] += jnp.sum(fm, axis=(0, 1)).reshape(1, F)
    acc_sq[...] += jnp.sum(fm * feats, axis=(0, 1)).reshape(1, F)
    acc_n[...] += jnp.sum(lens.astype(jnp.float32)).reshape(1, 1)

    @pl.when(i == pl.num_programs(0) - 1)
    def _():
        inv_n = 1.0 / acc_n[0, 0]
        mean = acc_sum[...] * inv_n
        var = jnp.maximum(acc_sq[...] * inv_n - mean * mean, 0.0)
        s = gamma_ref[...] * lax.rsqrt(var + EPS)
        scale_ref[...] = s
        bias_ref[...] = beta_ref[...] - mean * s


def _stats(feats3, lengths2, gamma2, beta2):
    return pl.pallas_call(
        _stats_body,
        grid=(NBLK,),
        in_specs=[
            pl.BlockSpec((BG, 1), lambda i: (i, 0)),
            pl.BlockSpec((BG, T, F), lambda i: (i, 0, 0)),
            pl.BlockSpec((1, F), lambda i: (0, 0)),
            pl.BlockSpec((1, F), lambda i: (0, 0)),
        ],
        out_specs=[
            pl.BlockSpec((1, F), lambda i: (0, 0)),
            pl.BlockSpec((1, F), lambda i: (0, 0)),
        ],
        out_shape=[
            jax.ShapeDtypeStruct((1, F), jnp.float32),
            jax.ShapeDtypeStruct((1, F), jnp.float32),
        ],
        scratch_shapes=[
            pltpu.VMEM((1, F), jnp.float32),
            pltpu.VMEM((1, F), jnp.float32),
            pltpu.VMEM((1, 1), jnp.float32),
        ],
    )(lengths2, feats3, gamma2, beta2)


def _norm_body(len_ref, feats_ref, scale_ref, bias_ref, out_ref):
    lens = len_ref[...]
    m3 = (lax.broadcasted_iota(jnp.int32, (BG, T, F), 1)
          < lens.reshape(BG, 1, 1)).astype(jnp.float32)
    s = scale_ref[...].reshape(1, 1, F)
    b = bias_ref[...].reshape(1, 1, F)
    out_ref[...] = (feats_ref[...] * s + b) * m3


def _norm(feats3, lengths2, scale, bias):
    return pl.pallas_call(
        _norm_body,
        grid=(NBLK,),
        in_specs=[
            pl.BlockSpec((BG, 1), lambda i: (i, 0)),
            pl.BlockSpec((BG, T, F), lambda i: (i, 0, 0)),
            pl.BlockSpec((1, F), lambda i: (0, 0)),
            pl.BlockSpec((1, F), lambda i: (0, 0)),
        ],
        out_specs=pl.BlockSpec((BG, T, F), lambda i: (i, 0, 0)),
        out_shape=jax.ShapeDtypeStruct((B, T, F), jnp.float32),
    )(lengths2, feats3, scale, bias)


def kernel(numeric, emb_idx, lengths, tables, gamma, beta):
    flat_tables = tables.reshape(N_EMB * VOCAB, EMB_DIM)
    offs = (jnp.arange(N_EMB, dtype=jnp.int32) * VOCAB)[None, :]
    flat_idx = (emb_idx.reshape(BT, N_EMB).astype(jnp.int32) + offs).reshape(-1)
    emb_flat = _sc_gather(flat_tables, flat_idx)
    emb3 = emb_flat.reshape(B, T, N_EMB * EMB_DIM)
    feats3 = jnp.concatenate([numeric, emb3], axis=2)
    lengths2 = lengths.reshape(B, 1).astype(jnp.int32)
    gamma2 = gamma.reshape(1, F)
    beta2 = beta.reshape(1, F)
    scale, bias = _stats(feats3, lengths2, gamma2, beta2)
    return _norm(feats3, lengths2, scale, bias)


# speedup vs baseline: 9.2779x; 1.0787x over previous
import functools

import jax
import jax.numpy as jnp
from jax import lax
from jax.experimental import pallas as pl
from jax.experimental.pallas import tpu as pltpu
from jax.experimental.pallas import tpu_sc as plsc

B, T = 1024, 50
N_NUM, N_EMB = 13, 26
EMB_DIM = 32
VOCAB = 100000
F = N_NUM + N_EMB * EMB_DIM
EPS = 1e-5

BT = B * T

NC, NS = 2, 16
NW = NC * NS
TOKW = BT // NW


N_IDX = BT * N_EMB
PER_W = N_IDX // NW
CHUNK = 1600
N_CH = PER_W // CHUNK


def _sc_gather(flat_tables, flat_idx):
    mesh = plsc.VectorSubcoreMesh(core_axis_name="c", subcore_axis_name="s")

    @functools.partial(
        pl.kernel,
        mesh=mesh,
        out_type=jax.ShapeDtypeStruct((N_IDX, EMB_DIM), jnp.float32),
        scratch_types=[
            pltpu.VMEM((2, CHUNK), jnp.int32),
            pltpu.VMEM((2, CHUNK, EMB_DIM), jnp.float32),
            pltpu.SemaphoreType.DMA,
            pltpu.SemaphoreType.DMA,
        ],
        compiler_params=pltpu.CompilerParams(use_tc_tiling_on_sc=False),
    )
    def gather_k(tbl_hbm, idx_hbm, out_hbm, idx_v, rows_v, sem_a, sem_b):
        wid = lax.axis_index("s") * NC + lax.axis_index("c")
        base = wid * PER_W
        sems = (sem_a, sem_b)

        def start(c, b):
            off = base + c * CHUNK
            pltpu.sync_copy(idx_hbm.at[pl.ds(off, CHUNK)], idx_v.at[b])
            pltpu.async_copy(tbl_hbm.at[idx_v.at[b]], rows_v.at[b], sems[b])

        def finish(c, b):
            off = base + c * CHUNK
            pltpu.make_async_copy(
                tbl_hbm.at[idx_v.at[b]], rows_v.at[b], sems[b]).wait()
            pltpu.sync_copy(rows_v.at[b], out_hbm.at[pl.ds(off, CHUNK)])

        start(0, 0)

        def pair(g, _):
            c0 = g * 2
            start(c0 + 1, 1)
            finish(c0, 0)

            @pl.when(c0 + 2 < N_CH)
            def _():
                start(c0 + 2, 0)

            finish(c0 + 1, 1)
            return 0

        lax.fori_loop(0, N_CH // 2, pair, 0)

    return gather_k(flat_tables, flat_idx)


BG = 16
NBLK = B // BG


def _stats_body(len_ref, feats_ref, gamma_ref, beta_ref,
                scale_ref, bias_ref, acc_sum, acc_sq, acc_n):
    i = pl.program_id(0)

    @pl.when(i == 0)
    def _():
        acc_sum[...] = jnp.zeros_like(acc_sum)
        acc_sq[...] = jnp.zeros_like(acc_sq)
        acc_n[...] = jnp.zeros_like(acc_n)

    lens = len_ref[...]
    m3 = (lax.broadcasted_iota(jnp.int32, (BG, T, F), 1)
          < lens.reshape(BG, 1, 1)).astype(jnp.float32)
    feats = feats_ref[...]
    fm = feats * m3
    acc_sum[...] += jnp.sum(fm, axis=(0, 1)).reshape(1, F)
    acc_sq[...] += jnp.sum(fm * feats, axis=(0, 1)).reshape(1, F)
    acc_n[...] += jnp.sum(lens.astype(jnp.float32)).reshape(1, 1)

    @pl.when(i == pl.num_programs(0) - 1)
    def _():
        inv_n = 1.0 / acc_n[0, 0]
        mean = acc_sum[...] * inv_n
        var = jnp.maximum(acc_sq[...] * inv_n - mean * mean, 0.0)
        s = gamma_ref[...] * lax.rsqrt(var + EPS)
        scale_ref[...] = s
        bias_ref[...] = beta_ref[...] - mean * s


def _stats(feats3, lengths2, gamma2, beta2):
    return pl.pallas_call(
        _stats_body,
        grid=(NBLK,),
        in_specs=[
            pl.BlockSpec((BG, 1), lambda i: (i, 0)),
            pl.BlockSpec((BG, T, F), lambda i: (i, 0, 0)),
            pl.BlockSpec((1, F), lambda i: (0, 0)),
            pl.BlockSpec((1, F), lambda i: (0, 0)),
        ],
        out_specs=[
            pl.BlockSpec((1, F), lambda i: (0, 0)),
            pl.BlockSpec((1, F), lambda i: (0, 0)),
        ],
        out_shape=[
            jax.ShapeDtypeStruct((1, F), jnp.float32),
            jax.ShapeDtypeStruct((1, F), jnp.float32),
        ],
        scratch_shapes=[
            pltpu.VMEM((1, F), jnp.float32),
            pltpu.VMEM((1, F), jnp.float32),
            pltpu.VMEM((1, 1), jnp.float32),
        ],
    )(lengths2, feats3, gamma2, beta2)


BGB = 128
TB = 10


def _norm_body(len_ref, feats_ref, scale_ref, bias_ref, out_ref):
    it = pl.program_id(1)
    lens = len_ref[...]
    s = jnp.swapaxes(scale_ref[...], 0, 1)
    bb = jnp.swapaxes(bias_ref[...], 0, 1)
    for k in range(TB):
        t = it * TB + k
        m_t = jnp.swapaxes((t < lens).astype(jnp.float32), 0, 1)
        xt = jnp.swapaxes(feats_ref[:, t, :], 0, 1)
        out_ref[k, :, :] = (xt * s + bb) * m_t


def _norm(feats3, lengths2, scale, bias):
    out_t = pl.pallas_call(
        _norm_body,
        grid=(B // BGB, T // TB),
        in_specs=[
            pl.BlockSpec((BGB, 1), lambda ib, it: (ib, 0)),
            pl.BlockSpec((BGB, T, F), lambda ib, it: (ib, 0, 0)),
            pl.BlockSpec((1, F), lambda ib, it: (0, 0)),
            pl.BlockSpec((1, F), lambda ib, it: (0, 0)),
        ],
        out_specs=pl.BlockSpec((TB, F, BGB), lambda ib, it: (it, 0, ib)),
        out_shape=jax.ShapeDtypeStruct((T, F, B), jnp.float32),
        compiler_params=pltpu.CompilerParams(
            vmem_limit_bytes=100 * 1024 * 1024),
    )(lengths2, feats3, scale, bias)
    return jnp.transpose(out_t, (2, 0, 1))


def kernel(numeric, emb_idx, lengths, tables, gamma, beta):
    flat_tables = tables.reshape(N_EMB * VOCAB, EMB_DIM)
    offs = (jnp.arange(N_EMB, dtype=jnp.int32) * VOCAB)[None, :]
    flat_idx = (emb_idx.reshape(BT, N_EMB).astype(jnp.int32) + offs).reshape(-1)
    emb_flat = _sc_gather(flat_tables, flat_idx)
    emb3 = emb_flat.reshape(B, T, N_EMB * EMB_DIM)
    feats3 = jnp.concatenate([numeric, emb3], axis=2)
    lengths2 = lengths.reshape(B, 1).astype(jnp.int32)
    gamma2 = gamma.reshape(1, F)
    beta2 = beta.reshape(1, F)
    scale, bias = _stats(feats3, lengths2, gamma2, beta2)
    return _norm(feats3, lengths2, scale, bias)


# speedup vs baseline: 9.7456x; 1.0504x over previous
import functools

import jax
import jax.numpy as jnp
from jax import lax
from jax.experimental import pallas as pl
from jax.experimental.pallas import tpu as pltpu
from jax.experimental.pallas import tpu_sc as plsc

B, T = 1024, 50
N_NUM, N_EMB = 13, 26
EMB_DIM = 32
VOCAB = 100000
F = N_NUM + N_EMB * EMB_DIM
EPS = 1e-5

BT = B * T

NC, NS = 2, 16
NW = NC * NS
TOKW = BT // NW


N_IDX = BT * N_EMB
PER_W = N_IDX // NW
CHUNK = 1600
N_CH = PER_W // CHUNK


def _sc_gather(flat_tables, flat_idx):
    mesh = plsc.VectorSubcoreMesh(core_axis_name="c", subcore_axis_name="s")

    @functools.partial(
        pl.kernel,
        mesh=mesh,
        out_type=jax.ShapeDtypeStruct((N_IDX, EMB_DIM), jnp.float32),
        scratch_types=[
            pltpu.VMEM((2, CHUNK), jnp.int32),
            pltpu.VMEM((2, CHUNK, EMB_DIM), jnp.float32),
            pltpu.SemaphoreType.DMA,
            pltpu.SemaphoreType.DMA,
        ],
        compiler_params=pltpu.CompilerParams(use_tc_tiling_on_sc=False),
    )
    def gather_k(tbl_hbm, idx_hbm, out_hbm, idx_v, rows_v, sem_a, sem_b):
        wid = lax.axis_index("s") * NC + lax.axis_index("c")
        base = wid * PER_W
        sems = (sem_a, sem_b)

        def start(c, b):
            off = base + c * CHUNK
            pltpu.sync_copy(idx_hbm.at[pl.ds(off, CHUNK)], idx_v.at[b])
            pltpu.async_copy(tbl_hbm.at[idx_v.at[b]], rows_v.at[b], sems[b])

        def finish(c, b):
            off = base + c * CHUNK
            pltpu.make_async_copy(
                tbl_hbm.at[idx_v.at[b]], rows_v.at[b], sems[b]).wait()
            pltpu.sync_copy(rows_v.at[b], out_hbm.at[pl.ds(off, CHUNK)])

        start(0, 0)

        def pair(g, _):
            c0 = g * 2
            start(c0 + 1, 1)
            finish(c0, 0)

            @pl.when(c0 + 2 < N_CH)
            def _():
                start(c0 + 2, 0)

            finish(c0 + 1, 1)
            return 0

        lax.fori_loop(0, N_CH // 2, pair, 0)

    return gather_k(flat_tables, flat_idx)


BG = 16
NBLK = B // BG


FE = N_EMB * EMB_DIM


def _stats_body(len_ref, num_ref, emb_ref, gn_ref, ge_ref, bn_ref, be_ref,
                sn_ref, se_ref, cn_ref, ce_ref,
                acc_sn, acc_qn, acc_se, acc_qe, acc_n):
    i = pl.program_id(0)

    @pl.when(i == 0)
    def _():
        acc_sn[...] = jnp.zeros_like(acc_sn)
        acc_qn[...] = jnp.zeros_like(acc_qn)
        acc_se[...] = jnp.zeros_like(acc_se)
        acc_qe[...] = jnp.zeros_like(acc_qe)
        acc_n[...] = jnp.zeros_like(acc_n)

    lens = len_ref[...]
    m3 = (lax.broadcasted_iota(jnp.int32, (BG, T, 1), 1)
          < lens.reshape(BG, 1, 1)).astype(jnp.float32)
    num = num_ref[...]
    emb = emb_ref[...]
    nm = num * m3
    em = emb * m3
    acc_sn[...] += jnp.sum(nm, axis=(0, 1)).reshape(1, N_NUM)
    acc_qn[...] += jnp.sum(nm * num, axis=(0, 1)).reshape(1, N_NUM)
    acc_se[...] += jnp.sum(em, axis=(0, 1)).reshape(1, FE)
    acc_qe[...] += jnp.sum(em * emb, axis=(0, 1)).reshape(1, FE)
    acc_n[...] += jnp.sum(lens.astype(jnp.float32)).reshape(1, 1)

    @pl.when(i == pl.num_programs(0) - 1)
    def _():
        inv_n = 1.0 / acc_n[0, 0]
        mean_n = acc_sn[...] * inv_n
        var_n = jnp.maximum(acc_qn[...] * inv_n - mean_n * mean_n, 0.0)
        s_n = gn_ref[...] * lax.rsqrt(var_n + EPS)
        sn_ref[...] = s_n
        cn_ref[...] = bn_ref[...] - mean_n * s_n
        mean_e = acc_se[...] * inv_n
        var_e = jnp.maximum(acc_qe[...] * inv_n - mean_e * mean_e, 0.0)
        s_e = ge_ref[...] * lax.rsqrt(var_e + EPS)
        se_ref[...] = s_e
        ce_ref[...] = be_ref[...] - mean_e * s_e


def _stats(numeric, emb3, lengths2, gamma_n, gamma_e, beta_n, beta_e):
    return pl.pallas_call(
        _stats_body,
        grid=(NBLK,),
        in_specs=[
            pl.BlockSpec((BG, 1), lambda i: (i, 0)),
            pl.BlockSpec((BG, T, N_NUM), lambda i: (i, 0, 0)),
            pl.BlockSpec((BG, T, FE), lambda i: (i, 0, 0)),
            pl.BlockSpec((1, N_NUM), lambda i: (0, 0)),
            pl.BlockSpec((1, FE), lambda i: (0, 0)),
            pl.BlockSpec((1, N_NUM), lambda i: (0, 0)),
            pl.BlockSpec((1, FE), lambda i: (0, 0)),
        ],
        out_specs=[
            pl.BlockSpec((1, N_NUM), lambda i: (0, 0)),
            pl.BlockSpec((1, FE), lambda i: (0, 0)),
            pl.BlockSpec((1, N_NUM), lambda i: (0, 0)),
            pl.BlockSpec((1, FE), lambda i: (0, 0)),
        ],
        out_shape=[
            jax.ShapeDtypeStruct((1, N_NUM), jnp.float32),
            jax.ShapeDtypeStruct((1, FE), jnp.float32),
            jax.ShapeDtypeStruct((1, N_NUM), jnp.float32),
            jax.ShapeDtypeStruct((1, FE), jnp.float32),
        ],
        scratch_shapes=[
            pltpu.VMEM((1, N_NUM), jnp.float32),
            pltpu.VMEM((1, N_NUM), jnp.float32),
            pltpu.VMEM((1, FE), jnp.float32),
            pltpu.VMEM((1, FE), jnp.float32),
            pltpu.VMEM((1, 1), jnp.float32),
        ],
    )(lengths2, numeric, emb3, gamma_n, gamma_e, beta_n, beta_e)


BGB = 128
TB = 5


def _norm_body(len_ref, num_ref, emb_ref, sn_ref, se_ref, cn_ref, ce_ref,
               out_ref):
    it = pl.program_id(1)
    lens = len_ref[...]
    s_n = jnp.swapaxes(sn_ref[...], 0, 1)
    c_n = jnp.swapaxes(cn_ref[...], 0, 1)
    s_e = jnp.swapaxes(se_ref[...], 0, 1)
    c_e = jnp.swapaxes(ce_ref[...], 0, 1)
    for k in range(TB):
        t = it * TB + k
        m_t = jnp.swapaxes((t < lens).astype(jnp.float32), 0, 1)
        et = jnp.swapaxes(emb_ref[:, t, :], 0, 1)
        out_ref[k, N_NUM:F, :] = (et * s_e + c_e) * m_t
        nt = jnp.swapaxes(num_ref[:, t, :], 0, 1)
        out_ref[k, 0:N_NUM, :] = (nt * s_n + c_n) * m_t


def _norm(numeric, emb3, lengths2, s_n, s_e, c_n, c_e):
    out_t = pl.pallas_call(
        _norm_body,
        grid=(B // BGB, T // TB),
        in_specs=[
            pl.BlockSpec((BGB, 1), lambda ib, it: (ib, 0)),
            pl.BlockSpec((BGB, T, N_NUM), lambda ib, it: (ib, 0, 0)),
            pl.BlockSpec((BGB, T, FE), lambda ib, it: (ib, 0, 0)),
            pl.BlockSpec((1, N_NUM), lambda ib, it: (0, 0)),
            pl.BlockSpec((1, FE), lambda ib, it: (0, 0)),
            pl.BlockSpec((1, N_NUM), lambda ib, it: (0, 0)),
            pl.BlockSpec((1, FE), lambda ib, it: (0, 0)),
        ],
        out_specs=pl.BlockSpec((TB, F, BGB), lambda ib, it: (it, 0, ib)),
        out_shape=jax.ShapeDtypeStruct((T, F, B), jnp.float32),
        compiler_params=pltpu.CompilerParams(
            vmem_limit_bytes=100 * 1024 * 1024),
    )(lengths2, numeric, emb3, s_n, s_e, c_n, c_e)
    return jnp.transpose(out_t, (2, 0, 1))


def kernel(numeric, emb_idx, lengths, tables, gamma, beta):
    flat_tables = tables.reshape(N_EMB * VOCAB, EMB_DIM)
    offs = (jnp.arange(N_EMB, dtype=jnp.int32) * VOCAB)[None, :]
    flat_idx = (emb_idx.reshape(BT, N_EMB).astype(jnp.int32) + offs).reshape(-1)
    emb_flat = _sc_gather(flat_tables, flat_idx)
    emb3 = emb_flat.reshape(B, T, FE)
    lengths2 = lengths.reshape(B, 1).astype(jnp.int32)
    gamma_n = gamma[:N_NUM].reshape(1, N_NUM)
    gamma_e = gamma[N_NUM:].reshape(1, FE)
    beta_n = beta[:N_NUM].reshape(1, N_NUM)
    beta_e = beta[N_NUM:].reshape(1, FE)
    s_n, s_e, c_n, c_e = _stats(numeric, emb3, lengths2,
                                gamma_n, gamma_e, beta_n, beta_e)
    return _norm(numeric, emb3, lengths2, s_n, s_e, c_n, c_e)


# speedup vs baseline: 11.5987x; 1.1901x over previous
import functools

import jax
import jax.numpy as jnp
from jax import lax
from jax.experimental import pallas as pl
from jax.experimental.pallas import tpu as pltpu
from jax.experimental.pallas import tpu_sc as plsc

B, T = 1024, 50
N_NUM, N_EMB = 13, 26
EMB_DIM = 32
VOCAB = 100000
F = N_NUM + N_EMB * EMB_DIM
FE = N_EMB * EMB_DIM
EPS = 1e-5
BT = B * T

NC, NS = 2, 16
NW = NC * NS
RPW = FE // NW
TCK = 6400
NTC = BT // TCK


def _sc_gather(tables_2d, idx_tm):
    mesh = plsc.VectorSubcoreMesh(core_axis_name="c", subcore_axis_name="s")

    @functools.partial(
        pl.kernel,
        mesh=mesh,
        out_type=jax.ShapeDtypeStruct((FE, BT), jnp.float32),
        scratch_types=[
            pltpu.VMEM((VOCAB,), jnp.float32),
            pltpu.VMEM((TCK,), jnp.int32),
            pltpu.VMEM((TCK,), jnp.float32),
        ],
        compiler_params=pltpu.CompilerParams(use_tc_tiling_on_sc=False,
                                             needs_layout_passes=False),
    )
    def gather_k(tbl_hbm, idx_hbm, out_hbm, row_v, idx_v, out_v):
        wid = lax.axis_index("s") * NC + lax.axis_index("c")

        def rbody(rr, _):
            rf = wid * RPW + rr
            j = rf // EMB_DIM
            pltpu.sync_copy(tbl_hbm.at[rf], row_v)

            def cbody(c, _):
                pltpu.sync_copy(idx_hbm.at[j, pl.ds(c * TCK, TCK)], idx_v)

                def gbody(g, _):
                    base = g * 128
                    for u in range(8):
                        iv = idx_v[pl.ds(base + u * 16, 16)]
                        out_v[pl.ds(base + u * 16, 16)] = (
                            plsc.load_gather(row_v, [iv]))
                    return 0

                lax.fori_loop(0, TCK // 128, gbody, 0)
                pltpu.sync_copy(out_v, out_hbm.at[rf, pl.ds(c * TCK, TCK)])
                return 0

            lax.fori_loop(0, NTC, cbody, 0)
            return 0

        lax.fori_loop(0, RPW, rbody, 0)

    return gather_k(tables_2d, idx_tm)


CK = 2048
NBC = BT // CK


def _stats_body(len_ref, num_ref, mask_ref, emb_ref,
                gn_ref, ge_ref, bn_ref, be_ref,
                sn_ref, se_ref, cn_ref, ce_ref,
                acc_sn, acc_qn, acc_se, acc_qe, acc_n):
    i = pl.program_id(0)

    @pl.when(i == 0)
    def _():
        lens = len_ref[...]
        lf = lens.astype(jnp.float32)
        acc_n[...] = jnp.sum(lf).reshape(1, 1)
        m3 = (lax.broadcasted_iota(jnp.int32, (T, 1, B), 0)
              < lens.reshape(1, 1, B)).astype(jnp.float32)
        num = num_ref[...]
        nm = num * m3
        acc_sn[...] = jnp.sum(nm, axis=(0, 2)).reshape(N_NUM, 1)
        acc_qn[...] = jnp.sum(nm * num, axis=(0, 2)).reshape(N_NUM, 1)
        acc_se[...] = jnp.zeros_like(acc_se)
        acc_qe[...] = jnp.zeros_like(acc_qe)

    emb = emb_ref[...]
    em = emb * mask_ref[...]
    acc_se[...] += jnp.sum(em, axis=1).reshape(FE, 1)
    acc_qe[...] += jnp.sum(em * emb, axis=1).reshape(FE, 1)

    @pl.when(i == pl.num_programs(0) - 1)
    def _():
        inv_n = 1.0 / acc_n[0, 0]
        mean_n = acc_sn[...] * inv_n
        var_n = jnp.maximum(acc_qn[...] * inv_n - mean_n * mean_n, 0.0)
        s_n = gn_ref[...] * lax.rsqrt(var_n + EPS)
        sn_ref[...] = s_n
        cn_ref[...] = bn_ref[...] - mean_n * s_n
        mean_e = acc_se[...] * inv_n
        var_e = jnp.maximum(acc_qe[...] * inv_n - mean_e * mean_e, 0.0)
        s_e = ge_ref[...] * lax.rsqrt(var_e + EPS)
        se_ref[...] = s_e
        ce_ref[...] = be_ref[...] - mean_e * s_e


def _stats(len_row, numeric_t, mask_tm, emb_fm,
           gamma_n, gamma_e, beta_n, beta_e):
    return pl.pallas_call(
        _stats_body,
        grid=(NBC,),
        in_specs=[
            pl.BlockSpec((1, B), lambda i: (0, 0)),
            pl.BlockSpec((T, N_NUM, B), lambda i: (0, 0, 0)),
            pl.BlockSpec((1, CK), lambda i: (0, i)),
            pl.BlockSpec((FE, CK), lambda i: (0, i)),
            pl.BlockSpec((N_NUM, 1), lambda i: (0, 0)),
            pl.BlockSpec((FE, 1), lambda i: (0, 0)),
            pl.BlockSpec((N_NUM, 1), lambda i: (0, 0)),
            pl.BlockSpec((FE, 1), lambda i: (0, 0)),
        ],
        out_specs=[
            pl.BlockSpec((N_NUM, 1), lambda i: (0, 0)),
            pl.BlockSpec((FE, 1), lambda i: (0, 0)),
            pl.BlockSpec((N_NUM, 1), lambda i: (0, 0)),
            pl.BlockSpec((FE, 1), lambda i: (0, 0)),
        ],
        out_shape=[
            jax.ShapeDtypeStruct((N_NUM, 1), jnp.float32),
            jax.ShapeDtypeStruct((FE, 1), jnp.float32),
            jax.ShapeDtypeStruct((N_NUM, 1), jnp.float32),
            jax.ShapeDtypeStruct((FE, 1), jnp.float32),
        ],
        scratch_shapes=[
            pltpu.VMEM((N_NUM, 1), jnp.float32),
            pltpu.VMEM((N_NUM, 1), jnp.float32),
            pltpu.VMEM((FE, 1), jnp.float32),
            pltpu.VMEM((FE, 1), jnp.float32),
            pltpu.VMEM((1, 1), jnp.float32),
        ],
    )(len_row, numeric_t, mask_tm, emb_fm, gamma_n, gamma_e, beta_n, beta_e)


TB = 1


def _norm_body(len_ref, num_ref, emb_ref, sn_ref, se_ref, cn_ref, ce_ref,
               out_ref):
    it = pl.program_id(0)
    lens = len_ref[...]
    s_n, c_n = sn_ref[...], cn_ref[...]
    s_e, c_e = se_ref[...], ce_ref[...]
    for k in range(TB):
        t = it * TB + k
        m = (t < lens).astype(jnp.float32)
        e = emb_ref[:, k * B:(k + 1) * B]
        out_ref[k, N_NUM:F, :] = (e * s_e + c_e) * m
        nt = num_ref[k]
        out_ref[k, 0:N_NUM, :] = (nt * s_n + c_n) * m


def _norm(len_row, numeric_t, emb_fm, s_n, s_e, c_n, c_e):
    out_t = pl.pallas_call(
        _norm_body,
        grid=(T // TB,),
        in_specs=[
            pl.BlockSpec((1, B), lambda it: (0, 0)),
            pl.BlockSpec((TB, N_NUM, B), lambda it: (it, 0, 0)),
            pl.BlockSpec((FE, TB * B), lambda it: (0, it)),
            pl.BlockSpec((N_NUM, 1), lambda it: (0, 0)),
            pl.BlockSpec((FE, 1), lambda it: (0, 0)),
            pl.BlockSpec((N_NUM, 1), lambda it: (0, 0)),
            pl.BlockSpec((FE, 1), lambda it: (0, 0)),
        ],
        out_specs=pl.BlockSpec((TB, F, B), lambda it: (it, 0, 0)),
        out_shape=jax.ShapeDtypeStruct((T, F, B), jnp.float32),
        compiler_params=pltpu.CompilerParams(
            vmem_limit_bytes=100 * 1024 * 1024),
    )(len_row, numeric_t, emb_fm, s_n, s_e, c_n, c_e)
    return jnp.transpose(out_t, (2, 0, 1))


def kernel(numeric, emb_idx, lengths, tables, gamma, beta):
    tables_2d = jnp.transpose(tables, (0, 2, 1)).reshape(FE, VOCAB)
    idx_tm = jnp.transpose(emb_idx, (2, 1, 0)).reshape(N_EMB, BT)
    idx_tm = idx_tm.astype(jnp.int32)
    numeric_t = jnp.transpose(numeric, (1, 2, 0))
    len_row = lengths.reshape(1, B).astype(jnp.int32)
    mask_tm = (jnp.arange(T, dtype=jnp.int32)[:, None]
               < lengths[None, :]).astype(jnp.float32).reshape(1, BT)
    emb_fm = _sc_gather(tables_2d, idx_tm)
    gamma_n = gamma[:N_NUM].reshape(N_NUM, 1)
    gamma_e = gamma[N_NUM:].reshape(FE, 1)
    beta_n = beta[:N_NUM].reshape(N_NUM, 1)
    beta_e = beta[N_NUM:].reshape(FE, 1)
    s_n, s_e, c_n, c_e = _stats(len_row, numeric_t, mask_tm, emb_fm,
                                gamma_n, gamma_e, beta_n, beta_e)
    return _norm(len_row, numeric_t, emb_fm, s_n, s_e, c_n, c_e)
